# Initial kernel scaffold; baseline (speedup 1.0000x reference)
import jax
import jax.numpy as jnp
from jax.experimental import pallas as pl


def kernel(x, embeddings):
    raise NotImplementedError("write your pallas kernel here")


# speedup vs baseline: 1.5634x; 1.5634x over previous
import functools

import jax
import jax.numpy as jnp
from jax import lax
from jax.experimental import pallas as pl
from jax.experimental.pallas import tpu as pltpu
from jax.experimental.pallas import tpu_sc as plsc

EMBED_DIM = 32
CHUNK = 128
K = 8
NUM_WORKERS = 32


def _build_gather(total_rows: int):
    n_chunks = total_rows // CHUNK
    cpw = n_chunks // NUM_WORKERS
    gpw = cpw // K
    rows_per_worker = cpw * CHUNK

    mesh = plsc.VectorSubcoreMesh(core_axis_name="c", subcore_axis_name="s")

    @functools.partial(
        pl.kernel,
        mesh=mesh,
        compiler_params=pltpu.CompilerParams(use_tc_tiling_on_sc=False),
        out_type=jax.ShapeDtypeStruct((total_rows, EMBED_DIM), jnp.float32),
        scratch_types=[
            pltpu.VMEM((cpw, CHUNK), jnp.int32),
            pltpu.VMEM((K * CHUNK, EMBED_DIM), jnp.float32),
            pltpu.SemaphoreType.DMA,
        ],
    )
    def gather_kernel(idx_hbm, table_hbm, out_hbm, idx_v, rows_v, gsem):
        wid = lax.axis_index("s") * 2 + lax.axis_index("c")
        cbase = wid * cpw
        pltpu.sync_copy(idx_hbm.at[pl.ds(cbase, cpw)], idx_v)

        def group_body(g, _):
            copies = []
            for j in range(K):
                copies.append(
                    pltpu.async_copy(
                        table_hbm.at[idx_v.at[g * K + j]],
                        rows_v.at[pl.ds(j * CHUNK, CHUNK)],
                        gsem,
                    )
                )
            for c in copies:
                c.wait()
            pltpu.sync_copy(
                rows_v,
                out_hbm.at[pl.ds((cbase + g * K) * CHUNK, K * CHUNK)],
            )
            return 0

        lax.fori_loop(0, gpw, group_body, 0)

    return gather_kernel


def kernel(x, embeddings):
    batch, n_fields = x.shape
    total = batch * n_fields
    idx2d = x.reshape(total).astype(jnp.int32).reshape(total // CHUNK, CHUNK)
    out = _build_gather(total)(idx2d, embeddings)
    return out.reshape(batch, n_fields, EMBED_DIM)
